# baseline (speedup 1.0000x reference)
import jax
import jax.numpy as jnp
from jax import lax
from jax.experimental import pallas as pl
from jax.experimental.pallas import tpu as pltpu
from jax.experimental.pallas import tpu_sc as plsc

DELTA_V = 0.3
DELTA_D = 1.5
ALPHA = 1.0
BETA = 1.0
GAMMA = 0.001
K = 24
KP = 32
D = 32
B = 4
N = 65536
NC, NS, L = 2, 16, 16
TPB = NS // 2
PPW = N // TPB
CHUNK = 1024
NCHUNK = PPW // CHUNK
GROUPS = CHUNK // L
ROW = K * D + KP
ROWP = 1024

_MESH = plsc.VectorSubcoreMesh(
    core_axis_name="c", subcore_axis_name="s", num_cores=NC, num_subcores=NS)


def _zero_ref(ref, n):
    def body(i, _):
        ref[pl.ds(pl.multiple_of(i * L, L), L)] = jnp.zeros((L,), ref.dtype)
        return 0
    lax.fori_loop(0, n // L, body, 0)


def _loss_body(emb_hbm, lab_hbm, out_mu, out_cnt, out_hs,
               sums, cnt_loc, hs_loc, mubuf, redbuf, wrow,
               shared, eb0, eb1, lb0, lb1, se0, se1, sl0, sl1):
    c = lax.axis_index("c")
    s = lax.axis_index("s")
    b = 2 * c + s // TPB
    p0 = (s % TPB) * PPW
    ebase = (b * N + p0) * D
    lbase = b * N + p0

    _zero_ref(sums, K * D)
    _zero_ref(cnt_loc, KP * L)
    _zero_ref(hs_loc, KP * L)

    lane = lax.iota(jnp.int32, L)
    lane32 = lane * D
    ones = jnp.ones((L,), jnp.float32)

    ebufs, lbufs, esems, lsems = (eb0, eb1), (lb0, lb1), (se0, se1), (sl0, sl1)

    def start(ch):
        i = ch % 2
        he = pltpu.async_copy(
            emb_hbm.at[pl.ds(pl.multiple_of(ebase + ch * CHUNK * D, 8),
                             CHUNK * D)], ebufs[i], esems[i])
        hl = pltpu.async_copy(
            lab_hbm.at[pl.ds(pl.multiple_of(lbase + ch * CHUNK, 8),
                             CHUNK)], lbufs[i], lsems[i])
        return he, hl

    def process_sum(ch):
        eb, lb = ebufs[ch % 2], lbufs[ch % 2]

        def grp(g, _):
            goff = pl.multiple_of(g * L, L)
            lbl = lb[pl.ds(goff, L)]
            plsc.addupdate_scatter(cnt_loc, [lbl * L + lane], ones)
            for j in range(L):
                kb = jnp.take(lbl, jnp.full((L,), j, jnp.int32))
                idx = lax.shift_left(kb, 5) + lane
                off = pl.multiple_of(goff * D + j * D, 8)
                plsc.addupdate_scatter(sums, [idx], eb[pl.ds(off, L)])
                plsc.addupdate_scatter(sums, [idx + L],
                                       eb[pl.ds(off + L, L)])
            return 0
        lax.fori_loop(0, GROUPS, grp, 0)

    pend = start(0)
    for ch in range(NCHUNK):
        for h in pend:
            h.wait()
        if ch + 1 < NCHUNK:
            pend = start(ch + 1)
        process_sum(ch)

    for kk in range(KP // L):
        kv = (lax.iota(jnp.int32, L) + kk * L) * L
        acc = jnp.zeros((L,), jnp.float32)
        for l in range(L):
            acc = acc + plsc.load_gather(cnt_loc, [kv + l])
        wrow[pl.ds(K * D + kk * L, L)] = acc

    def cpy(i, _):
        o = pl.multiple_of(i * L, L)
        wrow[pl.ds(o, L)] = sums[pl.ds(o, L)]
        return 0
    lax.fori_loop(0, K * D // L, cpy, 0)
    pltpu.sync_copy(wrow, shared.at[s])
    plsc.subcore_barrier()
    g0 = (s // TPB) * TPB
    for t in range(TPB):
        pltpu.sync_copy(shared.at[g0 + t], eb0.at[pl.ds(t * ROWP, ROWP)])
    plsc.subcore_barrier()

    def red(i, _):
        o = pl.multiple_of(i * L, L)
        a = eb0[pl.ds(o, L)]
        for t in range(1, TPB):
            a = a + eb0[pl.ds(o + t * ROWP, L)]
        redbuf[pl.ds(o, L)] = a
        return 0
    lax.fori_loop(0, ROW // L, red, 0)

    inv = [1.0 / redbuf[pl.ds(K * D + kk * L, L)] for kk in range(KP // L)]
    for k in range(K):
        ib = jnp.take(inv[k // L], jnp.full((L,), k % L, jnp.int32))
        o = k * D
        mubuf[pl.ds(o, L)] = redbuf[pl.ds(o, L)] * ib
        mubuf[pl.ds(o + L, L)] = redbuf[pl.ds(o + L, L)] * ib

    @pl.when(s % TPB == 0)
    def _():
        pltpu.sync_copy(mubuf, out_mu.at[pl.ds(b * (K * D), K * D)])
        pltpu.sync_copy(redbuf.at[pl.ds(K * D, KP)],
                        out_cnt.at[pl.ds(b * KP, KP)])

    def process_hinge(ch):
        eb, lb = ebufs[ch % 2], lbufs[ch % 2]

        def grp(g, _):
            goff = pl.multiple_of(g * L, L)
            lbl = lb[pl.ds(goff, L)]
            pbase = lane32 + g * (L * D)
            delta = lax.shift_left(lbl, 5) - pbase
            acc = [jnp.zeros((L,), jnp.float32) for _ in range(4)]
            dperm = lane
            for d in range(D):
                idx = pbase + dperm
                v = plsc.load_gather(eb, [idx])
                m = plsc.load_gather(mubuf, [idx + delta])
                t = v - m
                acc[d % 4] = acc[d % 4] + t * t
                dperm = (dperm + 1) & (D - 1)
            sq = (acc[0] + acc[1]) + (acc[2] + acc[3])
            iy = jnp.int32(0x5F3759DF) - lax.shift_right_logical(
                plsc.bitcast(sq, jnp.int32), 1)
            y = plsc.bitcast(iy, jnp.float32)
            half = 0.5 * sq
            for _ in range(3):
                y = y * (1.5 - half * y * y)
            dist = sq * y
            h = jnp.maximum(dist - DELTA_V, 0.0)
            plsc.addupdate_scatter(hs_loc, [lbl * L + lane], h * h)
            return 0
        lax.fori_loop(0, GROUPS, grp, 0)

    pend = start(0)
    for ch in range(NCHUNK):
        for h in pend:
            h.wait()
        if ch + 1 < NCHUNK:
            pend = start(ch + 1)
        process_hinge(ch)

    for kk in range(KP // L):
        kv = (lax.iota(jnp.int32, L) + kk * L) * L
        acc = jnp.zeros((L,), jnp.float32)
        for l in range(L):
            acc = acc + plsc.load_gather(hs_loc, [kv + l])
        wrow[pl.ds(kk * L, L)] = acc
    plsc.subcore_barrier()
    pltpu.sync_copy(wrow, shared.at[s])
    plsc.subcore_barrier()

    @pl.when(s % TPB == 0)
    def _():
        for t in range(TPB):
            pltpu.sync_copy(shared.at[g0 + t], eb0.at[pl.ds(t * ROWP, ROWP)])
        a0 = eb0[pl.ds(0, L)]
        a1 = eb0[pl.ds(L, L)]
        for t in range(1, TPB):
            a0 = a0 + eb0[pl.ds(t * ROWP, L)]
            a1 = a1 + eb0[pl.ds(t * ROWP + L, L)]
        wrow[pl.ds(0, L)] = a0
        wrow[pl.ds(L, L)] = a1
        pltpu.sync_copy(wrow.at[pl.ds(0, KP)],
                        out_hs.at[pl.ds(b * KP, KP)])


_loss_call = pl.kernel(
    _loss_body,
    out_type=(jax.ShapeDtypeStruct((B * K * D,), jnp.float32),
              jax.ShapeDtypeStruct((B * KP,), jnp.float32),
              jax.ShapeDtypeStruct((B * KP,), jnp.float32)),
    mesh=_MESH,
    scratch_types=(
        pltpu.VMEM((K * D,), jnp.float32),
        pltpu.VMEM((KP * L,), jnp.float32),
        pltpu.VMEM((KP * L,), jnp.float32),
        pltpu.VMEM((K * D,), jnp.float32),
        pltpu.VMEM((ROWP,), jnp.float32),
        pltpu.VMEM((ROWP,), jnp.float32),
        pltpu.VMEM_SHARED((NS, ROWP), jnp.float32),
        pltpu.VMEM((CHUNK * D,), jnp.float32),
        pltpu.VMEM((CHUNK * D,), jnp.float32),
        pltpu.VMEM((CHUNK,), jnp.int32),
        pltpu.VMEM((CHUNK,), jnp.int32),
        pltpu.SemaphoreType.DMA,
        pltpu.SemaphoreType.DMA,
        pltpu.SemaphoreType.DMA,
        pltpu.SemaphoreType.DMA,
    ),
    compiler_params=pltpu.CompilerParams(needs_layout_passes=False),
    name="disc_loss_sc",
)


def _final_tc(mu_ref, cnt_ref, hs_ref, out_ref):
    total = jnp.float32(0.0)
    eye = (lax.broadcasted_iota(jnp.int32, (K, K), 0)
           == lax.broadcasted_iota(jnp.int32, (K, K), 1))
    for b in range(B):
        mu = mu_ref[b]
        cnt = cnt_ref[b]
        hs = hs_ref[b]
        l_var = jnp.mean(hs / cnt)
        sq = jnp.sum((mu[:, None, :] - mu[None, :, :]) ** 2, axis=-1)
        dist = jnp.sqrt(jnp.where(eye, 1.0, sq))
        dh = jnp.maximum(2.0 * DELTA_D - dist, 0.0) ** 2
        dh = jnp.where(eye, 0.0, dh)
        l_dist = jnp.sum(dh) / (K * (K - 1))
        l_reg = jnp.mean(jnp.sqrt(jnp.sum(mu * mu, axis=1)))
        total = total + ALPHA * l_var + BETA * l_dist + GAMMA * l_reg
    out_ref[:, :] = jnp.reshape(total / B, (1, 1))


_final_call = pl.pallas_call(
    _final_tc,
    out_shape=jax.ShapeDtypeStruct((1, 1), jnp.float32),
)


def kernel(embeddings, instance_labels):
    emb_flat = embeddings.reshape(-1)
    lab_flat = instance_labels.reshape(-1)

    mu_r, cnt_r, hs_r = _loss_call(emb_flat, lab_flat)
    mu = mu_r.reshape(B, K, D)
    cnts = cnt_r.reshape(B, KP)[:, :K]
    hsum = hs_r.reshape(B, KP)[:, :K]

    return _final_call(mu, cnts, hsum)[0, 0]


# speedup vs baseline: 1.0279x; 1.0279x over previous
import jax
import jax.numpy as jnp
from jax import lax
from jax.experimental import pallas as pl
from jax.experimental.pallas import tpu as pltpu
from jax.experimental.pallas import tpu_sc as plsc

DELTA_V = 0.3
DELTA_D = 1.5
ALPHA = 1.0
BETA = 1.0
GAMMA = 0.001
K = 24
KP = 32
D = 32
B = 4
N = 65536
NC, NS, L = 2, 16, 16
TPB = NS // 2
PPW = N // TPB
CHUNK = 1024
NCHUNK = PPW // CHUNK
GROUPS = CHUNK // L
ROW = K * D + KP
ROWP = 1024

_MESH = plsc.VectorSubcoreMesh(
    core_axis_name="c", subcore_axis_name="s", num_cores=NC, num_subcores=NS)


def _zero_ref(ref, n):
    def body(i, _):
        ref[pl.ds(pl.multiple_of(i * L, L), L)] = jnp.zeros((L,), ref.dtype)
        return 0
    lax.fori_loop(0, n // L, body, 0)


def _loss_body(emb_hbm, lab_hbm, out_mu, out_cnt, out_hs,
               sums_loc, cnt_loc, hs_loc, mubuf, redbuf, wrow,
               shared, eb0, eb1, lb0, lb1, se0, se1, sl0, sl1):
    c = lax.axis_index("c")
    s = lax.axis_index("s")
    b = 2 * c + s // TPB
    p0 = (s % TPB) * PPW
    ebase = (b * N + p0) * D
    lbase = b * N + p0

    _zero_ref(sums_loc, L * K * D)
    _zero_ref(cnt_loc, KP * L)
    _zero_ref(hs_loc, KP * L)

    lane = lax.iota(jnp.int32, L)
    lane32 = lane * D
    ones = jnp.ones((L,), jnp.float32)

    ebufs, lbufs, esems, lsems = (eb0, eb1), (lb0, lb1), (se0, se1), (sl0, sl1)

    def start(ch):
        i = ch % 2
        he = pltpu.async_copy(
            emb_hbm.at[pl.ds(pl.multiple_of(ebase + ch * CHUNK * D, 8),
                             CHUNK * D)], ebufs[i], esems[i])
        hl = pltpu.async_copy(
            lab_hbm.at[pl.ds(pl.multiple_of(lbase + ch * CHUNK, 8),
                             CHUNK)], lbufs[i], lsems[i])
        return he, hl

    def process_sum(ch):
        eb, lb = ebufs[ch % 2], lbufs[ch % 2]

        def grp(g, _):
            goff = pl.multiple_of(g * L, L)
            lbl = lb[pl.ds(goff, L)]
            plsc.addupdate_scatter(cnt_loc, [lbl * L + lane], ones)
            sbase = lane * (K * D) + lax.shift_left(lbl, 5)
            pbase = lane32 + g * (L * D)
            dperm = lane
            for _d in range(D):
                v = plsc.load_gather(eb, [pbase + dperm])
                plsc.addupdate_scatter(sums_loc, [sbase + dperm], v)
                dperm = (dperm + 1) & (D - 1)
            return 0
        lax.fori_loop(0, GROUPS, grp, 0)

    pend = start(0)
    for ch in range(NCHUNK):
        for h in pend:
            h.wait()
        if ch + 1 < NCHUNK:
            pend = start(ch + 1)
        process_sum(ch)

    for kk in range(KP // L):
        kv = (lax.iota(jnp.int32, L) + kk * L) * L
        acc = jnp.zeros((L,), jnp.float32)
        for l in range(L):
            acc = acc + plsc.load_gather(cnt_loc, [kv + l])
        wrow[pl.ds(K * D + kk * L, L)] = acc

    def cpy(i, _):
        o = pl.multiple_of(i * L, L)
        a = sums_loc[pl.ds(o, L)]
        for r in range(1, L):
            a = a + sums_loc[pl.ds(o + r * (K * D), L)]
        wrow[pl.ds(o, L)] = a
        return 0
    lax.fori_loop(0, K * D // L, cpy, 0)
    pltpu.sync_copy(wrow, shared.at[s])
    plsc.subcore_barrier()
    g0 = (s // TPB) * TPB
    for t in range(TPB):
        pltpu.sync_copy(shared.at[g0 + t], eb0.at[pl.ds(t * ROWP, ROWP)])
    plsc.subcore_barrier()

    def red(i, _):
        o = pl.multiple_of(i * L, L)
        a = eb0[pl.ds(o, L)]
        for t in range(1, TPB):
            a = a + eb0[pl.ds(o + t * ROWP, L)]
        redbuf[pl.ds(o, L)] = a
        return 0
    lax.fori_loop(0, ROW // L, red, 0)

    inv = [1.0 / redbuf[pl.ds(K * D + kk * L, L)] for kk in range(KP // L)]
    for k in range(K):
        ib = jnp.take(inv[k // L], jnp.full((L,), k % L, jnp.int32))
        o = k * D
        mubuf[pl.ds(o, L)] = redbuf[pl.ds(o, L)] * ib
        mubuf[pl.ds(o + L, L)] = redbuf[pl.ds(o + L, L)] * ib

    @pl.when(s % TPB == 0)
    def _():
        pltpu.sync_copy(mubuf, out_mu.at[pl.ds(b * (K * D), K * D)])
        pltpu.sync_copy(redbuf.at[pl.ds(K * D, KP)],
                        out_cnt.at[pl.ds(b * KP, KP)])

    def process_hinge(ch):
        eb, lb = ebufs[ch % 2], lbufs[ch % 2]

        def grp(g, _):
            goff = pl.multiple_of(g * L, L)
            lbl = lb[pl.ds(goff, L)]
            pbase = lane32 + g * (L * D)
            delta = lax.shift_left(lbl, 5) - pbase
            acc = [jnp.zeros((L,), jnp.float32) for _ in range(4)]
            dperm = lane
            for d in range(D):
                idx = pbase + dperm
                v = plsc.load_gather(eb, [idx])
                m = plsc.load_gather(mubuf, [idx + delta])
                t = v - m
                acc[d % 4] = acc[d % 4] + t * t
                dperm = (dperm + 1) & (D - 1)
            sq = (acc[0] + acc[1]) + (acc[2] + acc[3])
            iy = jnp.int32(0x5F3759DF) - lax.shift_right_logical(
                plsc.bitcast(sq, jnp.int32), 1)
            y = plsc.bitcast(iy, jnp.float32)
            half = 0.5 * sq
            for _ in range(3):
                y = y * (1.5 - half * y * y)
            dist = sq * y
            h = jnp.maximum(dist - DELTA_V, 0.0)
            plsc.addupdate_scatter(hs_loc, [lbl * L + lane], h * h)
            return 0
        lax.fori_loop(0, GROUPS, grp, 0)

    pend = start(0)
    for ch in range(NCHUNK):
        for h in pend:
            h.wait()
        if ch + 1 < NCHUNK:
            pend = start(ch + 1)
        process_hinge(ch)

    for kk in range(KP // L):
        kv = (lax.iota(jnp.int32, L) + kk * L) * L
        acc = jnp.zeros((L,), jnp.float32)
        for l in range(L):
            acc = acc + plsc.load_gather(hs_loc, [kv + l])
        wrow[pl.ds(kk * L, L)] = acc
    plsc.subcore_barrier()
    pltpu.sync_copy(wrow, shared.at[s])
    plsc.subcore_barrier()

    @pl.when(s % TPB == 0)
    def _():
        for t in range(TPB):
            pltpu.sync_copy(shared.at[g0 + t], eb0.at[pl.ds(t * ROWP, ROWP)])
        a0 = eb0[pl.ds(0, L)]
        a1 = eb0[pl.ds(L, L)]
        for t in range(1, TPB):
            a0 = a0 + eb0[pl.ds(t * ROWP, L)]
            a1 = a1 + eb0[pl.ds(t * ROWP + L, L)]
        wrow[pl.ds(0, L)] = a0
        wrow[pl.ds(L, L)] = a1
        pltpu.sync_copy(wrow.at[pl.ds(0, KP)],
                        out_hs.at[pl.ds(b * KP, KP)])


_loss_call = pl.kernel(
    _loss_body,
    out_type=(jax.ShapeDtypeStruct((B * K * D,), jnp.float32),
              jax.ShapeDtypeStruct((B * KP,), jnp.float32),
              jax.ShapeDtypeStruct((B * KP,), jnp.float32)),
    mesh=_MESH,
    scratch_types=(
        pltpu.VMEM((L * K * D,), jnp.float32),
        pltpu.VMEM((KP * L,), jnp.float32),
        pltpu.VMEM((KP * L,), jnp.float32),
        pltpu.VMEM((K * D,), jnp.float32),
        pltpu.VMEM((ROWP,), jnp.float32),
        pltpu.VMEM((ROWP,), jnp.float32),
        pltpu.VMEM_SHARED((NS, ROWP), jnp.float32),
        pltpu.VMEM((CHUNK * D,), jnp.float32),
        pltpu.VMEM((CHUNK * D,), jnp.float32),
        pltpu.VMEM((CHUNK,), jnp.int32),
        pltpu.VMEM((CHUNK,), jnp.int32),
        pltpu.SemaphoreType.DMA,
        pltpu.SemaphoreType.DMA,
        pltpu.SemaphoreType.DMA,
        pltpu.SemaphoreType.DMA,
    ),
    compiler_params=pltpu.CompilerParams(needs_layout_passes=False),
    name="disc_loss_sc",
)


def _final_tc(mu_ref, cnt_ref, hs_ref, out_ref):
    total = jnp.float32(0.0)
    eye = (lax.broadcasted_iota(jnp.int32, (K, K), 0)
           == lax.broadcasted_iota(jnp.int32, (K, K), 1))
    for b in range(B):
        mu = mu_ref[b]
        cnt = cnt_ref[b]
        hs = hs_ref[b]
        l_var = jnp.mean(hs / cnt)
        sq = jnp.sum((mu[:, None, :] - mu[None, :, :]) ** 2, axis=-1)
        dist = jnp.sqrt(jnp.where(eye, 1.0, sq))
        dh = jnp.maximum(2.0 * DELTA_D - dist, 0.0) ** 2
        dh = jnp.where(eye, 0.0, dh)
        l_dist = jnp.sum(dh) / (K * (K - 1))
        l_reg = jnp.mean(jnp.sqrt(jnp.sum(mu * mu, axis=1)))
        total = total + ALPHA * l_var + BETA * l_dist + GAMMA * l_reg
    out_ref[:, :] = jnp.reshape(total / B, (1, 1))


_final_call = pl.pallas_call(
    _final_tc,
    out_shape=jax.ShapeDtypeStruct((1, 1), jnp.float32),
)


def kernel(embeddings, instance_labels):
    emb_flat = embeddings.reshape(-1)
    lab_flat = instance_labels.reshape(-1)

    mu_r, cnt_r, hs_r = _loss_call(emb_flat, lab_flat)
    mu = mu_r.reshape(B, K, D)
    cnts = cnt_r.reshape(B, KP)[:, :K]
    hsum = hs_r.reshape(B, KP)[:, :K]

    return _final_call(mu, cnts, hsum)[0, 0]


# speedup vs baseline: 1.1180x; 1.0877x over previous
import functools

import jax
import jax.numpy as jnp
from jax import lax
from jax.experimental import pallas as pl
from jax.experimental.pallas import tpu as pltpu
from jax.experimental.pallas import tpu_sc as plsc

DELTA_V = 0.3
DELTA_D = 1.5
ALPHA = 1.0
BETA = 1.0
GAMMA = 0.001
K = 24
KP = 32
D = 32
B = 4
N = 65536
NC, NS, L = 2, 16, 16
NW = NC * NS
WPB = NW // B
PPW = N // WPB
CHUNK = 1024
NCHUNK = PPW // CHUNK
GROUPS = CHUNK // L

_MESH = plsc.VectorSubcoreMesh(
    core_axis_name="c", subcore_axis_name="s", num_cores=NC, num_subcores=NS)


def _wid():
    return lax.axis_index("s") * NC + lax.axis_index("c")


def _zero_ref(ref, n):
    def body(i, _):
        ref[pl.ds(pl.multiple_of(i * L, L), L)] = jnp.zeros((L,), ref.dtype)
        return 0
    lax.fori_loop(0, n // L, body, 0)


def _lane_reduce(src, dst, ncols):
    def body(j, _):
        col = pl.multiple_of(j * L, L)
        acc = src[pl.ds(col, L)]
        for r in range(1, L):
            acc = acc + src[pl.ds(col + r * ncols, L)]
        dst[pl.ds(col, L)] = acc
        return 0
    lax.fori_loop(0, ncols // L, body, 0)


def _sumsc_body(emb_hbm, lab_hbm, out_s, out_c,
                sums_loc, cnt_loc, sums_red, cnt_red,
                eb0, eb1, lb0, lb1, se0, se1, sl0, sl1):
    wid = _wid()
    b = wid // WPB
    p0 = (wid % WPB) * PPW
    ebase = (b * N + p0) * D
    lbase = b * N + p0

    _zero_ref(sums_loc, L * K * D)
    _zero_ref(cnt_loc, L * KP)

    lane = lax.iota(jnp.int32, L)
    lanebase = lane * (K * D)
    cntbase = lane * KP
    ones = jnp.ones((L,), jnp.float32)

    ebufs, lbufs, esems, lsems = (eb0, eb1), (lb0, lb1), (se0, se1), (sl0, sl1)

    def start(ch):
        i = ch % 2
        he = pltpu.async_copy(
            emb_hbm.at[pl.ds(pl.multiple_of(ebase + ch * CHUNK * D, 8),
                             CHUNK * D)], ebufs[i], esems[i])
        hl = pltpu.async_copy(
            lab_hbm.at[pl.ds(pl.multiple_of(lbase + ch * CHUNK, 8),
                             CHUNK)], lbufs[i], lsems[i])
        return he, hl

    def process(ch):
        i = ch % 2
        eb, lb = ebufs[i], lbufs[i]

        def one(g):
            goff = pl.multiple_of(g * L, L)
            lbl = lb[pl.ds(goff, L)]
            plsc.addupdate_scatter(cnt_loc, [cntbase + lbl], ones)
            sbase = lanebase + lbl * D
            pbase = g * (L * D) + lane * D
            dperm = lane
            for _ in range(D):
                v = plsc.load_gather(eb, [pbase + dperm])
                plsc.addupdate_scatter(sums_loc, [sbase + dperm], v)
                dperm = (dperm + 1) & (D - 1)

        def grp(g, _):
            one(2 * g)
            one(2 * g + 1)
            return 0
        lax.fori_loop(0, GROUPS // 2, grp, 0)

    pend = start(0)
    for ch in range(NCHUNK):
        for h in pend:
            h.wait()
        if ch + 1 < NCHUNK:
            pend = start(ch + 1)
        process(ch)

    _lane_reduce(sums_loc, sums_red, K * D)
    _lane_reduce(cnt_loc, cnt_red, KP)
    pltpu.sync_copy(sums_red, out_s.at[wid])
    pltpu.sync_copy(cnt_red, out_c.at[wid])


def _hinge_body(emb_hbm, lab_hbm, mu_hbm, out_h,
                hs_loc, hs_red, mubuf,
                eb0, eb1, lb0, lb1, se0, se1, sl0, sl1):
    wid = _wid()
    b = wid // WPB
    p0 = (wid % WPB) * PPW
    ebase = (b * N + p0) * D
    lbase = b * N + p0

    pltpu.sync_copy(mu_hbm.at[pl.ds(pl.multiple_of(b * K * D, 8), K * D)],
                    mubuf)
    _zero_ref(hs_loc, L * KP)

    lane = lax.iota(jnp.int32, L)
    hbase = lane * KP

    ebufs, lbufs, esems, lsems = (eb0, eb1), (lb0, lb1), (se0, se1), (sl0, sl1)

    def start(ch):
        i = ch % 2
        he = pltpu.async_copy(
            emb_hbm.at[pl.ds(pl.multiple_of(ebase + ch * CHUNK * D, 8),
                             CHUNK * D)], ebufs[i], esems[i])
        hl = pltpu.async_copy(
            lab_hbm.at[pl.ds(pl.multiple_of(lbase + ch * CHUNK, 8),
                             CHUNK)], lbufs[i], lsems[i])
        return he, hl

    def process(ch):
        i = ch % 2
        eb, lb = ebufs[i], lbufs[i]

        def one(g):
            goff = pl.multiple_of(g * L, L)
            lbl = lb[pl.ds(goff, L)]
            mbase = lbl * D
            pbase = g * (L * D) + lane * D
            acc = [jnp.zeros((L,), jnp.float32) for _ in range(4)]
            dperm = lane
            for d in range(D):
                v = plsc.load_gather(eb, [pbase + dperm])
                m = plsc.load_gather(mubuf, [mbase + dperm])
                t = v - m
                acc[d % 4] = acc[d % 4] + t * t
                dperm = (dperm + 1) & (D - 1)
            s = (acc[0] + acc[1]) + (acc[2] + acc[3])
            iy = jnp.int32(0x5F3759DF) - lax.shift_right_logical(
                plsc.bitcast(s, jnp.int32), 1)
            y = plsc.bitcast(iy, jnp.float32)
            half_s = 0.5 * s
            for _ in range(3):
                y = y * (1.5 - half_s * y * y)
            dist = s * y
            h = jnp.maximum(dist - DELTA_V, 0.0)
            plsc.addupdate_scatter(hs_loc, [hbase + lbl], h * h)

        def grp(g, _):
            one(2 * g)
            one(2 * g + 1)
            return 0
        lax.fori_loop(0, GROUPS // 2, grp, 0)

    pend = start(0)
    for ch in range(NCHUNK):
        for h in pend:
            h.wait()
        if ch + 1 < NCHUNK:
            pend = start(ch + 1)
        process(ch)

    _lane_reduce(hs_loc, hs_red, KP)
    pltpu.sync_copy(hs_red, out_h.at[wid])


_sums_call = pl.kernel(
    _sumsc_body,
    out_type=(jax.ShapeDtypeStruct((NW, K * D), jnp.float32),
              jax.ShapeDtypeStruct((NW, KP), jnp.float32)),
    mesh=_MESH,
    scratch_types=(
        pltpu.VMEM((L * K * D,), jnp.float32),
        pltpu.VMEM((L * KP,), jnp.float32),
        pltpu.VMEM((K * D,), jnp.float32),
        pltpu.VMEM((KP,), jnp.float32),
        pltpu.VMEM((CHUNK * D,), jnp.float32),
        pltpu.VMEM((CHUNK * D,), jnp.float32),
        pltpu.VMEM((CHUNK,), jnp.int32),
        pltpu.VMEM((CHUNK,), jnp.int32),
        pltpu.SemaphoreType.DMA,
        pltpu.SemaphoreType.DMA,
        pltpu.SemaphoreType.DMA,
        pltpu.SemaphoreType.DMA,
    ),
    compiler_params=pltpu.CompilerParams(needs_layout_passes=False),
    name="disc_loss_segsum_sc",
)

_hinge_call = pl.kernel(
    _hinge_body,
    out_type=jax.ShapeDtypeStruct((NW, KP), jnp.float32),
    mesh=_MESH,
    scratch_types=(
        pltpu.VMEM((L * KP,), jnp.float32),
        pltpu.VMEM((KP,), jnp.float32),
        pltpu.VMEM((K * D,), jnp.float32),
        pltpu.VMEM((CHUNK * D,), jnp.float32),
        pltpu.VMEM((CHUNK * D,), jnp.float32),
        pltpu.VMEM((CHUNK,), jnp.int32),
        pltpu.VMEM((CHUNK,), jnp.int32),
        pltpu.SemaphoreType.DMA,
        pltpu.SemaphoreType.DMA,
        pltpu.SemaphoreType.DMA,
        pltpu.SemaphoreType.DMA,
    ),
    compiler_params=pltpu.CompilerParams(needs_layout_passes=False),
    name="disc_loss_hinge_sc",
)


def _final_tc(mu_ref, cnt_ref, hs_ref, out_ref):
    total = jnp.float32(0.0)
    eye = (lax.broadcasted_iota(jnp.int32, (K, K), 0)
           == lax.broadcasted_iota(jnp.int32, (K, K), 1))
    for b in range(B):
        mu = mu_ref[b]
        cnt = cnt_ref[b]
        hs = hs_ref[b]
        l_var = jnp.mean(hs / cnt)
        sq = jnp.sum((mu[:, None, :] - mu[None, :, :]) ** 2, axis=-1)
        dist = jnp.sqrt(jnp.where(eye, 1.0, sq))
        dh = jnp.maximum(2.0 * DELTA_D - dist, 0.0) ** 2
        dh = jnp.where(eye, 0.0, dh)
        l_dist = jnp.sum(dh) / (K * (K - 1))
        l_reg = jnp.mean(jnp.sqrt(jnp.sum(mu * mu, axis=1)))
        total = total + ALPHA * l_var + BETA * l_dist + GAMMA * l_reg
    out_ref[:, :] = jnp.reshape(total / B, (1, 1))


_final_call = pl.pallas_call(
    _final_tc,
    out_shape=jax.ShapeDtypeStruct((1, 1), jnp.float32),
)


def kernel(embeddings, instance_labels):
    emb_flat = embeddings.reshape(-1)
    lab_flat = instance_labels.reshape(-1)

    psums, pcnts = _sums_call(emb_flat, lab_flat)
    sums = psums.reshape(B, WPB, K, D).sum(1)
    cnts = pcnts.reshape(B, WPB, KP)[:, :, :K].sum(1)
    mu = sums / cnts[:, :, None]

    phs = _hinge_call(emb_flat, lab_flat, mu.reshape(-1))
    hsum = phs.reshape(B, WPB, KP)[:, :, :K].sum(1)

    return _final_call(mu, cnts, hsum)[0, 0]
